# Initial kernel scaffold; baseline (speedup 1.0000x reference)
import jax
import jax.numpy as jnp
from jax.experimental import pallas as pl


def kernel(x, edge_index, W1a, b1a, W1b, b1b, W2a, b2a, W2b, b2b, W3a, b3a, W3b, b3b, W4a, b4a, W4b, b4b):
    raise NotImplementedError("write your pallas kernel here")


# speedup vs baseline: 4.8843x; 4.8843x over previous
import functools

import jax
import jax.numpy as jnp
from jax import lax
from jax.experimental import pallas as pl
from jax.experimental.pallas import tpu as pltpu
from jax.experimental.pallas import tpu_sc as plsc

N = 100000
E = 1600000
D_IN = 30
H = 64

N_PAD = 100352
E_ROWS = 12800
E_PAD = E_ROWS * 128
ROWS_PER_TILE = E_ROWS // 16
KB = 8
BLOCKS = ROWS_PER_TILE // KB
N_ACC = 100016
STRIPE = N_ACC // 16
ZROWS = 640
ZFULL = STRIPE // ZROWS
ZREM = STRIPE - ZFULL * ZROWS

BN = 512
GRID_N = N_PAD // BN


def _agg_job(src_hbm, dst_hbm, table_hbm, out_hbm,
             src_v, dst_v, rows_v, zbuf, acc_sh, gsem, sid):
    for z in range(ZFULL):
        pltpu.sync_copy(zbuf, acc_sh.at[pl.ds(sid * STRIPE + z * ZROWS, ZROWS)])
    pltpu.sync_copy(zbuf.at[pl.ds(0, ZREM)],
                    acc_sh.at[pl.ds(sid * STRIPE + ZFULL * ZROWS, ZREM)])
    plsc.subcore_barrier()

    def block(i, carry):
        off = sid * ROWS_PER_TILE + i * KB
        pltpu.sync_copy(src_hbm.at[pl.ds(off, KB)], src_v)
        pltpu.sync_copy(dst_hbm.at[pl.ds(off, KB)], dst_v)
        descs = []
        for j in range(KB):
            descs.append(pltpu.async_copy(
                table_hbm.at[src_v.at[j]],
                rows_v.at[pl.ds(j * 128, 128)], gsem))
        for d in descs:
            d.wait()
        for j in range(KB):
            pltpu.sync_copy(rows_v.at[pl.ds(j * 128, 128)],
                            acc_sh.at[dst_v.at[j]], add=True)
        return carry

    lax.fori_loop(0, BLOCKS, block, 0)
    plsc.subcore_barrier()
    pltpu.sync_copy(acc_sh.at[pl.ds(sid * STRIPE, STRIPE)],
                    out_hbm.at[pl.ds(sid * STRIPE, STRIPE)])


def _make_agg1():
    mesh = plsc.VectorSubcoreMesh(core_axis_name="c", subcore_axis_name="s")

    @functools.partial(
        pl.kernel,
        out_type=[jax.ShapeDtypeStruct((N_PAD, 16), jnp.float32)
                  for _ in range(2)],
        mesh=mesh,
        compiler_params=pltpu.CompilerParams(use_tc_tiling_on_sc=False),
        scratch_types=[
            pltpu.VMEM((KB, 128), jnp.int32),
            pltpu.VMEM((KB, 128), jnp.int32),
            pltpu.VMEM((KB * 128, 16), jnp.float32),
            pltpu.VMEM((ZROWS, 16), jnp.float32),
            pltpu.VMEM_SHARED((N_ACC, 16), jnp.float32),
            pltpu.SemaphoreType.DMA,
        ],
    )
    def agg1(src_hbm, dst_hbm, zin_hbm, x0_hbm, x1_hbm, a0_hbm, a1_hbm,
             src_v, dst_v, rows_v, zbuf, acc_sh, gsem):
        cid = lax.axis_index("c")
        sid = lax.axis_index("s")
        pltpu.sync_copy(zin_hbm, zbuf)

        @pl.when(cid == 0)
        def _():
            _agg_job(src_hbm, dst_hbm, x0_hbm, a0_hbm,
                     src_v, dst_v, rows_v, zbuf, acc_sh, gsem, sid)

        @pl.when(cid == 1)
        def _():
            _agg_job(src_hbm, dst_hbm, x1_hbm, a1_hbm,
                     src_v, dst_v, rows_v, zbuf, acc_sh, gsem, sid)

    return agg1


def _make_agg2():
    mesh = plsc.VectorSubcoreMesh(core_axis_name="c", subcore_axis_name="s")

    @functools.partial(
        pl.kernel,
        out_type=[jax.ShapeDtypeStruct((N_PAD, 16), jnp.float32)
                  for _ in range(8)],
        mesh=mesh,
        compiler_params=pltpu.CompilerParams(use_tc_tiling_on_sc=False),
        scratch_types=[
            pltpu.VMEM((KB, 128), jnp.int32),
            pltpu.VMEM((KB, 128), jnp.int32),
            pltpu.VMEM((KB * 128, 16), jnp.float32),
            pltpu.VMEM((ZROWS, 16), jnp.float32),
            pltpu.VMEM_SHARED((N_ACC, 16), jnp.float32),
            pltpu.SemaphoreType.DMA,
        ],
    )
    def agg2(src_hbm, dst_hbm, zin_hbm,
             z10_hbm, z11_hbm, z12_hbm, z13_hbm,
             z30_hbm, z31_hbm, z32_hbm, z33_hbm,
             a10_hbm, a11_hbm, a12_hbm, a13_hbm,
             a30_hbm, a31_hbm, a32_hbm, a33_hbm,
             src_v, dst_v, rows_v, zbuf, acc_sh, gsem):
        cid = lax.axis_index("c")
        sid = lax.axis_index("s")
        pltpu.sync_copy(zin_hbm, zbuf)

        @pl.when(cid == 0)
        def _():
            for tbl, out in ((z10_hbm, a10_hbm), (z11_hbm, a11_hbm),
                             (z12_hbm, a12_hbm), (z13_hbm, a13_hbm)):
                _agg_job(src_hbm, dst_hbm, tbl, out,
                         src_v, dst_v, rows_v, zbuf, acc_sh, gsem, sid)

        @pl.when(cid == 1)
        def _():
            for tbl, out in ((z30_hbm, a30_hbm), (z31_hbm, a31_hbm),
                             (z32_hbm, a32_hbm), (z33_hbm, a33_hbm)):
                _agg_job(src_hbm, dst_hbm, tbl, out,
                         src_v, dst_v, rows_v, zbuf, acc_sh, gsem, sid)

    return agg2


def _stage1_body(xp, a00, a01, W1a, b1a, W1b, b1b, W3a, b3a, W3b, b3b,
                 z10, z11, z12, z13, z30, z31, z32, z33):
    h = xp[...] + jnp.concatenate([a00[...], a01[...]], axis=1)
    t1 = jnp.maximum(jnp.dot(h, W1a[...],
                             preferred_element_type=jnp.float32) + b1a[...], 0.0)
    z1 = jnp.dot(t1, W1b[...], preferred_element_type=jnp.float32) + b1b[...]
    t3 = jnp.maximum(jnp.dot(h, W3a[...],
                             preferred_element_type=jnp.float32) + b3a[...], 0.0)
    z3 = jnp.dot(t3, W3b[...], preferred_element_type=jnp.float32) + b3b[...]
    for k, ref in enumerate((z10, z11, z12, z13)):
        ref[...] = z1[:, 16 * k:16 * (k + 1)]
    for k, ref in enumerate((z30, z31, z32, z33)):
        ref[...] = z3[:, 16 * k:16 * (k + 1)]


def _stage2_body(z10, z11, z12, z13, a10, a11, a12, a13,
                 z30, z31, z32, z33, a30, a31, a32, a33,
                 W2a, b2a, W2b, b2b, W4a, b4a, W4b, b4b,
                 zsrc, ztar):
    h1 = (jnp.concatenate([z10[...], z11[...], z12[...], z13[...]], axis=1)
          + jnp.concatenate([a10[...], a11[...], a12[...], a13[...]], axis=1))
    t1 = jnp.maximum(jnp.dot(h1, W2a[...],
                             preferred_element_type=jnp.float32) + b2a[...], 0.0)
    zsrc[...] = jnp.dot(t1, W2b[...],
                        preferred_element_type=jnp.float32) + b2b[...]
    h3 = (jnp.concatenate([z30[...], z31[...], z32[...], z33[...]], axis=1)
          + jnp.concatenate([a30[...], a31[...], a32[...], a33[...]], axis=1))
    t3 = jnp.maximum(jnp.dot(h3, W4a[...],
                             preferred_element_type=jnp.float32) + b4a[...], 0.0)
    ztar[...] = jnp.dot(t3, W4b[...],
                        preferred_element_type=jnp.float32) + b4b[...]


def _row_spec(w):
    return pl.BlockSpec((BN, w), lambda i: (i, 0))


def _full_spec(shape):
    return pl.BlockSpec(shape, lambda i: tuple(0 for _ in shape))


def kernel(x, edge_index, W1a, b1a, W1b, b1b, W2a, b2a, W2b, b2b,
           W3a, b3a, W3b, b3b, W4a, b4a, W4b, b4b):
    x = x.astype(jnp.float32)
    f32 = jnp.float32

    xp = jnp.pad(x, ((0, N_PAD - N), (0, 32 - D_IN)))
    x0, x1 = xp[:, :16], xp[:, 16:]
    src = jnp.concatenate(
        [edge_index[0], jnp.zeros((E_PAD - E,), jnp.int32)]).reshape(E_ROWS, 128)
    dst = jnp.concatenate(
        [edge_index[1], jnp.full((E_PAD - E,), N, jnp.int32)]).reshape(E_ROWS, 128)
    zin = jnp.zeros((ZROWS, 16), f32)
    W1a_p = jnp.pad(W1a, ((0, 2), (0, 0)))
    W3a_p = jnp.pad(W3a, ((0, 2), (0, 0)))
    b1a_r, b1b_r = b1a.reshape(1, H), b1b.reshape(1, H)
    b2a_r, b2b_r = b2a.reshape(1, H), b2b.reshape(1, H)
    b3a_r, b3b_r = b3a.reshape(1, H), b3b.reshape(1, H)
    b4a_r, b4b_r = b4a.reshape(1, H), b4b.reshape(1, H)

    a00, a01 = _make_agg1()(src, dst, zin, x0, x1)

    slice_shape = jax.ShapeDtypeStruct((N_PAD, 16), f32)
    stage1 = pl.pallas_call(
        _stage1_body,
        grid=(GRID_N,),
        in_specs=[_row_spec(32), _row_spec(16), _row_spec(16),
                  _full_spec((32, H)), _full_spec((1, H)),
                  _full_spec((H, H)), _full_spec((1, H)),
                  _full_spec((32, H)), _full_spec((1, H)),
                  _full_spec((H, H)), _full_spec((1, H))],
        out_specs=[_row_spec(16)] * 8,
        out_shape=[slice_shape] * 8,
    )
    z10, z11, z12, z13, z30, z31, z32, z33 = stage1(
        xp, a00, a01, W1a_p, b1a_r, W1b, b1b_r, W3a_p, b3a_r, W3b, b3b_r)

    (a10, a11, a12, a13, a30, a31, a32, a33) = _make_agg2()(
        src, dst, zin, z10, z11, z12, z13, z30, z31, z32, z33)

    out_shape = jax.ShapeDtypeStruct((N_PAD, H), f32)
    stage2 = pl.pallas_call(
        _stage2_body,
        grid=(GRID_N,),
        in_specs=[_row_spec(16)] * 16 + [
            _full_spec((H, H)), _full_spec((1, H)),
            _full_spec((H, H)), _full_spec((1, H)),
            _full_spec((H, H)), _full_spec((1, H)),
            _full_spec((H, H)), _full_spec((1, H))],
        out_specs=[_row_spec(H)] * 2,
        out_shape=[out_shape] * 2,
    )
    zsrc, ztar = stage2(
        z10, z11, z12, z13, a10, a11, a12, a13,
        z30, z31, z32, z33, a30, a31, a32, a33,
        W2a, b2a_r, W2b, b2b_r, W4a, b4a_r, W4b, b4b_r)

    return (zsrc[:N], ztar[:N])


# speedup vs baseline: 5.3702x; 1.0995x over previous
import functools

import jax
import jax.numpy as jnp
from jax import lax
from jax.experimental import pallas as pl
from jax.experimental.pallas import tpu as pltpu
from jax.experimental.pallas import tpu_sc as plsc

N = 100000
E = 1600000
D_IN = 30
H = 64

N_PAD = 100352
E_ROWS = 12800
E_PAD = E_ROWS * 128
ROWS_PER_TILE = E_ROWS // 16
N_ACC = 100016
STRIPE = N_ACC // 16
MBR = 16
RB = 4
BLK_PER_MB = MBR // RB
MB_PAIRS = ROWS_PER_TILE // (2 * MBR)

BN = 512
GRID_N = N_PAD // BN

_SC_SCRATCH = [
    pltpu.VMEM((MBR, 128), jnp.int32),
    pltpu.VMEM((MBR, 128), jnp.int32),
    pltpu.VMEM((MBR, 128), jnp.int32),
    pltpu.VMEM((MBR, 128), jnp.int32),
    pltpu.VMEM((RB, 128, 16), jnp.float32),
    pltpu.VMEM((RB, 128, 16), jnp.float32),
    pltpu.VMEM_SHARED((N_ACC, 16), jnp.float32),
    pltpu.SemaphoreType.DMA,
    pltpu.SemaphoreType.DMA,
    pltpu.SemaphoreType.DMA,
]


def _agg_job(src_hbm, dst_hbm, zin_hbm, table_hbm, out_hbm, bufs, sid):
    smA, dmA, smB, dmB, rows, ssems, gsem, acc_sh = bufs

    def drain(p):
        for r in range(RB):
            pltpu.make_async_copy(rows[p].at[r], acc_sh.at[dmA.at[r]],
                                  ssems[p]).wait()

    def do_mb(sm, dm, off, first):
        pltpu.sync_copy(src_hbm.at[pl.ds(off, MBR)], sm)
        pltpu.sync_copy(dst_hbm.at[pl.ds(off, MBR)], dm)
        for blk in range(BLK_PER_MB):
            p = blk % 2
            if not (first and blk < 2):
                drain(p)
            descs = [
                pltpu.async_copy(table_hbm.at[sm.at[blk * RB + r]],
                                 rows[p].at[r], gsem)
                for r in range(RB)]
            for d in descs:
                d.wait()
            for r in range(RB):
                pltpu.async_copy(rows[p].at[r],
                                 acc_sh.at[dm.at[blk * RB + r]],
                                 ssems[p], add=True)

    pltpu.sync_copy(zin_hbm, acc_sh.at[pl.ds(sid * STRIPE, STRIPE)])
    plsc.subcore_barrier()
    base = sid * ROWS_PER_TILE

    def mb_pair(k, carry):
        off_a = base + k * (2 * MBR)

        @pl.when(k > 0)
        def _():
            drain(0)
            drain(1)

        do_mb(smA, dmA, off_a, first=True)
        do_mb(smB, dmB, off_a + MBR, first=False)
        return carry

    lax.fori_loop(0, MB_PAIRS, mb_pair, 0)
    drain(0)
    drain(1)
    plsc.subcore_barrier()
    pltpu.sync_copy(acc_sh.at[pl.ds(sid * STRIPE, STRIPE)],
                    out_hbm.at[pl.ds(sid * STRIPE, STRIPE)])


def _make_agg1():
    mesh = plsc.VectorSubcoreMesh(core_axis_name="c", subcore_axis_name="s")

    @functools.partial(
        pl.kernel,
        out_type=[jax.ShapeDtypeStruct((N_PAD, 16), jnp.float32)
                  for _ in range(2)],
        mesh=mesh,
        compiler_params=pltpu.CompilerParams(use_tc_tiling_on_sc=False),
        scratch_types=_SC_SCRATCH,
    )
    def agg1(src_hbm, dst_hbm, zin_hbm, x0_hbm, x1_hbm, a0_hbm, a1_hbm,
             smA, dmA, smB, dmB, rows0, rows1, acc_sh, gsem, ssem0, ssem1):
        cid = lax.axis_index("c")
        sid = lax.axis_index("s")
        bufs = (smA, dmA, smB, dmB, (rows0, rows1), (ssem0, ssem1),
                gsem, acc_sh)

        @pl.when(cid == 0)
        def _():
            _agg_job(src_hbm, dst_hbm, zin_hbm, x0_hbm, a0_hbm, bufs, sid)

        @pl.when(cid == 1)
        def _():
            _agg_job(src_hbm, dst_hbm, zin_hbm, x1_hbm, a1_hbm, bufs, sid)

    return agg1


def _make_agg2():
    mesh = plsc.VectorSubcoreMesh(core_axis_name="c", subcore_axis_name="s")

    @functools.partial(
        pl.kernel,
        out_type=[jax.ShapeDtypeStruct((N_PAD, 16), jnp.float32)
                  for _ in range(8)],
        mesh=mesh,
        compiler_params=pltpu.CompilerParams(use_tc_tiling_on_sc=False),
        scratch_types=_SC_SCRATCH,
    )
    def agg2(src_hbm, dst_hbm, zin_hbm,
             z10_hbm, z11_hbm, z12_hbm, z13_hbm,
             z30_hbm, z31_hbm, z32_hbm, z33_hbm,
             a10_hbm, a11_hbm, a12_hbm, a13_hbm,
             a30_hbm, a31_hbm, a32_hbm, a33_hbm,
             smA, dmA, smB, dmB, rows0, rows1, acc_sh, gsem, ssem0, ssem1):
        cid = lax.axis_index("c")
        sid = lax.axis_index("s")
        bufs = (smA, dmA, smB, dmB, (rows0, rows1), (ssem0, ssem1),
                gsem, acc_sh)

        @pl.when(cid == 0)
        def _():
            for tbl, out in ((z10_hbm, a10_hbm), (z11_hbm, a11_hbm),
                             (z12_hbm, a12_hbm), (z13_hbm, a13_hbm)):
                _agg_job(src_hbm, dst_hbm, zin_hbm, tbl, out, bufs, sid)

        @pl.when(cid == 1)
        def _():
            for tbl, out in ((z30_hbm, a30_hbm), (z31_hbm, a31_hbm),
                             (z32_hbm, a32_hbm), (z33_hbm, a33_hbm)):
                _agg_job(src_hbm, dst_hbm, zin_hbm, tbl, out, bufs, sid)

    return agg2


def _stage1_body(xp, a00, a01, W1a, b1a, W1b, b1b, W3a, b3a, W3b, b3b,
                 z10, z11, z12, z13, z30, z31, z32, z33):
    h = xp[...] + jnp.concatenate([a00[...], a01[...]], axis=1)
    t1 = jnp.maximum(jnp.dot(h, W1a[...],
                             preferred_element_type=jnp.float32) + b1a[...], 0.0)
    z1 = jnp.dot(t1, W1b[...], preferred_element_type=jnp.float32) + b1b[...]
    t3 = jnp.maximum(jnp.dot(h, W3a[...],
                             preferred_element_type=jnp.float32) + b3a[...], 0.0)
    z3 = jnp.dot(t3, W3b[...], preferred_element_type=jnp.float32) + b3b[...]
    for k, ref in enumerate((z10, z11, z12, z13)):
        ref[...] = z1[:, 16 * k:16 * (k + 1)]
    for k, ref in enumerate((z30, z31, z32, z33)):
        ref[...] = z3[:, 16 * k:16 * (k + 1)]


def _stage2_body(z10, z11, z12, z13, a10, a11, a12, a13,
                 z30, z31, z32, z33, a30, a31, a32, a33,
                 W2a, b2a, W2b, b2b, W4a, b4a, W4b, b4b,
                 zsrc, ztar):
    h1 = (jnp.concatenate([z10[...], z11[...], z12[...], z13[...]], axis=1)
          + jnp.concatenate([a10[...], a11[...], a12[...], a13[...]], axis=1))
    t1 = jnp.maximum(jnp.dot(h1, W2a[...],
                             preferred_element_type=jnp.float32) + b2a[...], 0.0)
    zsrc[...] = jnp.dot(t1, W2b[...],
                        preferred_element_type=jnp.float32) + b2b[...]
    h3 = (jnp.concatenate([z30[...], z31[...], z32[...], z33[...]], axis=1)
          + jnp.concatenate([a30[...], a31[...], a32[...], a33[...]], axis=1))
    t3 = jnp.maximum(jnp.dot(h3, W4a[...],
                             preferred_element_type=jnp.float32) + b4a[...], 0.0)
    ztar[...] = jnp.dot(t3, W4b[...],
                        preferred_element_type=jnp.float32) + b4b[...]


def _row_spec(w):
    return pl.BlockSpec((BN, w), lambda i: (i, 0))


def _full_spec(shape):
    return pl.BlockSpec(shape, lambda i: tuple(0 for _ in shape))


def kernel(x, edge_index, W1a, b1a, W1b, b1b, W2a, b2a, W2b, b2b,
           W3a, b3a, W3b, b3b, W4a, b4a, W4b, b4b):
    x = x.astype(jnp.float32)
    f32 = jnp.float32

    xp = jnp.pad(x, ((0, N_PAD - N), (0, 32 - D_IN)))
    x0, x1 = xp[:, :16], xp[:, 16:]
    src = jnp.concatenate(
        [edge_index[0], jnp.zeros((E_PAD - E,), jnp.int32)]).reshape(E_ROWS, 128)
    dst = jnp.concatenate(
        [edge_index[1], jnp.full((E_PAD - E,), N, jnp.int32)]).reshape(E_ROWS, 128)
    zin = jnp.zeros((STRIPE, 16), f32)
    W1a_p = jnp.pad(W1a, ((0, 2), (0, 0)))
    W3a_p = jnp.pad(W3a, ((0, 2), (0, 0)))
    b1a_r, b1b_r = b1a.reshape(1, H), b1b.reshape(1, H)
    b2a_r, b2b_r = b2a.reshape(1, H), b2b.reshape(1, H)
    b3a_r, b3b_r = b3a.reshape(1, H), b3b.reshape(1, H)
    b4a_r, b4b_r = b4a.reshape(1, H), b4b.reshape(1, H)

    a00, a01 = _make_agg1()(src, dst, zin, x0, x1)

    slice_shape = jax.ShapeDtypeStruct((N_PAD, 16), f32)
    stage1 = pl.pallas_call(
        _stage1_body,
        grid=(GRID_N,),
        in_specs=[_row_spec(32), _row_spec(16), _row_spec(16),
                  _full_spec((32, H)), _full_spec((1, H)),
                  _full_spec((H, H)), _full_spec((1, H)),
                  _full_spec((32, H)), _full_spec((1, H)),
                  _full_spec((H, H)), _full_spec((1, H))],
        out_specs=[_row_spec(16)] * 8,
        out_shape=[slice_shape] * 8,
    )
    z10, z11, z12, z13, z30, z31, z32, z33 = stage1(
        xp, a00, a01, W1a_p, b1a_r, W1b, b1b_r, W3a_p, b3a_r, W3b, b3b_r)

    (a10, a11, a12, a13, a30, a31, a32, a33) = _make_agg2()(
        src, dst, zin, z10, z11, z12, z13, z30, z31, z32, z33)

    out_shape = jax.ShapeDtypeStruct((N_PAD, H), f32)
    stage2 = pl.pallas_call(
        _stage2_body,
        grid=(GRID_N,),
        in_specs=[_row_spec(16)] * 16 + [
            _full_spec((H, H)), _full_spec((1, H)),
            _full_spec((H, H)), _full_spec((1, H)),
            _full_spec((H, H)), _full_spec((1, H)),
            _full_spec((H, H)), _full_spec((1, H))],
        out_specs=[_row_spec(H)] * 2,
        out_shape=[out_shape] * 2,
    )
    zsrc, ztar = stage2(
        z10, z11, z12, z13, a10, a11, a12, a13,
        z30, z31, z32, z33, a30, a31, a32, a33,
        W2a, b2a_r, W2b, b2b_r, W4a, b4a_r, W4b, b4b_r)

    return (zsrc[:N], ztar[:N])
